# baseline (speedup 1.0000x reference)
import functools

import jax
import jax.numpy as jnp
from jax import lax
from jax.experimental import pallas as pl
from jax.experimental.pallas import tpu as pltpu
from jax.experimental.pallas import tpu_sc as plsc

_SEQ_LEN = 8192
_EMB_DIM = 1024
_N = 16384

_NC = 2
_NS = 16
_NW = _NC * _NS
_B_PER_W = _N // _NW
_CH = 56
_SIZES = [_CH] * (_B_PER_W // _CH) + (
    [_B_PER_W % _CH] if _B_PER_W % _CH else []
)
_OFFS = [sum(_SIZES[:i]) for i in range(len(_SIZES))]
_NCH = len(_SIZES)


@functools.partial(
    pl.kernel,
    out_type=jax.ShapeDtypeStruct((_N, _EMB_DIM), jnp.float32),
    mesh=plsc.VectorSubcoreMesh(core_axis_name="c", subcore_axis_name="s"),
    scratch_types=(
        [pltpu.VMEM((_B_PER_W,), jnp.int32)]
        + [pltpu.VMEM((_CH, _EMB_DIM), jnp.float32) for _ in range(2)]
        + [pltpu.SemaphoreType.DMA for _ in range(4)]
    ),
)
def _gather_rows(t_hbm, pem_hbm, out_hbm, idx_v, buf0, buf1, g0, g1, s0, s1):
    bufs = (buf0, buf1)
    gsems = (g0, g1)
    ssems = (s0, s1)

    wid = lax.axis_index("s") * _NC + lax.axis_index("c")
    base = wid * _B_PER_W
    pltpu.sync_copy(t_hbm.at[pl.ds(base, _B_PER_W)], idx_v)

    def start_gather(ch):
        b = ch % 2
        return pltpu.async_copy(
            pem_hbm.at[idx_v.at[pl.ds(_OFFS[ch], _SIZES[ch])]],
            bufs[b].at[pl.ds(0, _SIZES[ch])],
            gsems[b],
        )

    gathers = [None] * _NCH
    stores = [None] * _NCH
    gathers[0] = start_gather(0)
    for ch in range(_NCH):
        b = ch % 2
        gathers[ch].wait()
        if ch + 1 < _NCH:
            if ch >= 1:
                stores[ch - 1].wait()
            gathers[ch + 1] = start_gather(ch + 1)
        stores[ch] = pltpu.async_copy(
            bufs[b].at[pl.ds(0, _SIZES[ch])],
            out_hbm.at[pl.ds(base + _OFFS[ch], _SIZES[ch])],
            ssems[b],
        )
    stores[_NCH - 2].wait()
    stores[_NCH - 1].wait()


@jax.jit
def kernel(t, pem):
    return _gather_rows(t.astype(jnp.int32), pem)


# speedup vs baseline: 1.0270x; 1.0270x over previous
import functools

import jax
import jax.numpy as jnp
from jax import lax
from jax.experimental import pallas as pl
from jax.experimental.pallas import tpu as pltpu
from jax.experimental.pallas import tpu_sc as plsc

_SEQ_LEN = 8192
_EMB_DIM = 1024
_N = 16384

_NC = 2
_NS = 16
_NW = _NC * _NS
_B_PER_W = _N // _NW
_CH = 16
_NCH = _B_PER_W // _CH
_NBUF = 7


@functools.partial(
    pl.kernel,
    out_type=jax.ShapeDtypeStruct((_N, _EMB_DIM), jnp.float32),
    mesh=plsc.VectorSubcoreMesh(core_axis_name="c", subcore_axis_name="s"),
    scratch_types=(
        [pltpu.VMEM((_B_PER_W,), jnp.int32)]
        + [pltpu.VMEM((_CH, _EMB_DIM), jnp.float32) for _ in range(_NBUF)]
        + [pltpu.SemaphoreType.DMA for _ in range(2 * _NBUF)]
    ),
)
def _gather_rows(t_hbm, pem_hbm, out_hbm, idx_v, *rest):
    bufs = rest[:_NBUF]
    gsems = rest[_NBUF : 2 * _NBUF]
    ssems = rest[2 * _NBUF :]

    wid = lax.axis_index("s") * _NC + lax.axis_index("c")
    base = wid * _B_PER_W
    pltpu.sync_copy(t_hbm.at[pl.ds(base, _B_PER_W)], idx_v)

    def start_gather(ch):
        b = ch % _NBUF
        return pltpu.async_copy(
            pem_hbm.at[idx_v.at[pl.ds(ch * _CH, _CH)]], bufs[b], gsems[b]
        )

    gathers = [None] * _NCH
    stores = [None] * _NCH
    for ch in range(_NBUF - 1):
        gathers[ch] = start_gather(ch)
    for ch in range(_NCH):
        b = ch % _NBUF
        gathers[ch].wait()
        nxt = ch + _NBUF - 1
        if nxt < _NCH:
            prev = nxt - _NBUF
            if prev >= 0:
                stores[prev].wait()
            gathers[nxt] = start_gather(nxt)
        stores[ch] = pltpu.async_copy(
            bufs[b], out_hbm.at[pl.ds(base + ch * _CH, _CH)], ssems[b]
        )
    for ch in range(max(0, _NCH - _NBUF), _NCH):
        stores[ch].wait()


@jax.jit
def kernel(t, pem):
    return _gather_rows(t.astype(jnp.int32), pem)
